# Initial kernel scaffold; baseline (speedup 1.0000x reference)
import jax
import jax.numpy as jnp
from jax.experimental import pallas as pl


def kernel(inputs, table):
    raise NotImplementedError("write your pallas kernel here")


# speedup vs baseline: 1.0950x; 1.0950x over previous
import functools

import jax
import jax.numpy as jnp
from jax import lax
from jax.experimental import pallas as pl
from jax.experimental.pallas import tpu as pltpu
from jax.experimental.pallas import tpu_sc as plsc

B_TOTAL = 16384 * 50
D = 32

NC = 2
NS = 16
NW = NC * NS
B_PER_W = B_TOTAL // NW
CHUNK = 1024
NCHUNK = B_PER_W // CHUNK

_mesh = plsc.VectorSubcoreMesh(core_axis_name="c", subcore_axis_name="s")


@functools.partial(
    pl.kernel,
    mesh=_mesh,
    out_type=jax.ShapeDtypeStruct((B_TOTAL, D), jnp.float32),
    scratch_types=[
        pltpu.VMEM((CHUNK,), jnp.int32),
        pltpu.VMEM((CHUNK, D), jnp.float32),
        pltpu.SemaphoreType.DMA,
    ],
    compiler_params=pltpu.CompilerParams(use_tc_tiling_on_sc=False),
)
def _gather_kernel(idx_hbm, table_hbm, out_hbm, idx_v, rows_v, sem):
    wid = lax.axis_index("s") * NC + lax.axis_index("c")
    base = wid * B_PER_W

    def body(i, carry):
        off = base + i * CHUNK
        pltpu.sync_copy(idx_hbm.at[pl.ds(off, CHUNK)], idx_v)
        pltpu.async_copy(table_hbm.at[idx_v], rows_v, sem).wait()
        pltpu.sync_copy(rows_v, out_hbm.at[pl.ds(off, CHUNK)])
        return carry

    lax.fori_loop(0, NCHUNK, body, 0)


def kernel(inputs, table):
    idx = inputs.reshape(-1).astype(jnp.int32)
    out = _gather_kernel(idx, table)
    return out.reshape(inputs.shape + (table.shape[1],))


# speedup vs baseline: 1.1125x; 1.0160x over previous
import functools

import jax
import jax.numpy as jnp
from jax import lax
from jax.experimental import pallas as pl
from jax.experimental.pallas import tpu as pltpu
from jax.experimental.pallas import tpu_sc as plsc

B_TOTAL = 16384 * 50
D = 32

NC = 2
NS = 16
NW = NC * NS
B_PER_W = B_TOTAL // NW
CHUNK = 800
NCHUNK = B_PER_W // CHUNK
NBUF = 4
NGROUP = NCHUNK // NBUF

_mesh = plsc.VectorSubcoreMesh(core_axis_name="c", subcore_axis_name="s")


@functools.partial(
    pl.kernel,
    mesh=_mesh,
    out_type=jax.ShapeDtypeStruct((B_TOTAL, D), jnp.float32),
    scratch_types=(
        [pltpu.VMEM((CHUNK,), jnp.int32) for _ in range(NBUF)]
        + [pltpu.VMEM((NBUF, CHUNK, D), jnp.float32)]
        + [pltpu.SemaphoreType.DMA for _ in range(2 * NBUF)]
    ),
    compiler_params=pltpu.CompilerParams(use_tc_tiling_on_sc=False),
)
def _gather_kernel(idx_hbm, table_hbm, out_hbm, *refs):
    idxs = refs[:NBUF]
    rows_v = refs[NBUF]
    gsems = refs[NBUF + 1:2 * NBUF + 1]
    ssems = refs[2 * NBUF + 1:]

    wid = lax.axis_index("s") * NC + lax.axis_index("c")
    base = wid * B_PER_W

    def load_idx(c, b):
        pltpu.sync_copy(idx_hbm.at[pl.ds(base + c * CHUNK, CHUNK)], idxs[b])

    def gather_desc(b):
        return pltpu.make_async_copy(
            table_hbm.at[idxs[b]], rows_v.at[b], gsems[b])

    def store_desc(c, b):
        return pltpu.make_async_copy(
            rows_v.at[b], out_hbm.at[pl.ds(base + c * CHUNK, CHUNK)],
            ssems[b])

    for b in range(NBUF):
        load_idx(b, b)
        gather_desc(b).start()

    def outer(gi, carry):
        g = gi * NBUF
        for b in range(NBUF):
            c = g + b
            gather_desc(b).wait()
            store_desc(c, b).start()
        for b in range(NBUF):
            nc = g + NBUF + b

            @pl.when(nc < NCHUNK)
            def _():
                store_desc(nc - NBUF, b).wait()
                load_idx(nc, b)
                gather_desc(b).start()
        return carry

    lax.fori_loop(0, NGROUP, outer, 0)

    for b in range(NBUF):
        store_desc(NCHUNK - NBUF + b, b).wait()


def kernel(inputs, table):
    idx = inputs.reshape(-1).astype(jnp.int32)
    out = _gather_kernel(idx, table)
    return out.reshape(inputs.shape + (table.shape[1],))


# speedup vs baseline: 1.8067x; 1.6240x over previous
import functools

import jax
import jax.numpy as jnp
from jax import lax
from jax.experimental import pallas as pl
from jax.experimental.pallas import tpu as pltpu
from jax.experimental.pallas import tpu_sc as plsc

B_TOTAL = 16384 * 50
D = 32

NC = 2
NS = 16
NW = NC * NS
B_PER_W = B_TOTAL // NW
CHUNK = 800
NCHUNK = B_PER_W // CHUNK
NBUF = 4
NGROUP = NCHUNK // NBUF

_mesh = plsc.VectorSubcoreMesh(core_axis_name="c", subcore_axis_name="s")


@functools.partial(
    pl.kernel,
    mesh=_mesh,
    out_type=jax.ShapeDtypeStruct((16384, 50, D), jnp.float32),
    scratch_types=(
        [pltpu.VMEM((CHUNK,), jnp.int32) for _ in range(NBUF)]
        + [pltpu.VMEM((NBUF, CHUNK, D), jnp.float32)]
        + [pltpu.SemaphoreType.DMA for _ in range(2 * NBUF)]
    ),
    compiler_params=pltpu.CompilerParams(use_tc_tiling_on_sc=False),
)
def _gather_kernel(idx_hbm, table_hbm, out_hbm, *refs):
    idxs = refs[:NBUF]
    rows_v = refs[NBUF]
    gsems = refs[NBUF + 1:2 * NBUF + 1]
    ssems = refs[2 * NBUF + 1:]

    wid = lax.axis_index("s") * NC + lax.axis_index("c")
    base = wid * B_PER_W

    def load_idx(c, b):
        pltpu.sync_copy(idx_hbm.at[pl.ds(base + c * CHUNK, CHUNK)], idxs[b])

    def gather_desc(b):
        return pltpu.make_async_copy(
            table_hbm.at[idxs[b]], rows_v.at[b], gsems[b])

    ROWS_PER_CHUNK = CHUNK // 50
    row_base = wid * (B_PER_W // 50)

    def store_start(c, b):
        for r in range(ROWS_PER_CHUNK):
            pltpu.make_async_copy(
                rows_v.at[b].at[pl.ds(r * 50, 50)],
                out_hbm.at[row_base + c * ROWS_PER_CHUNK + r],
                ssems[b]).start()

    def store_wait(b):
        for r in range(ROWS_PER_CHUNK):
            pltpu.make_async_copy(
                rows_v.at[b].at[pl.ds(r * 50, 50)],
                out_hbm.at[row_base],
                ssems[b]).wait()

    for b in range(NBUF):
        load_idx(b, b)
        gather_desc(b).start()

    def outer(gi, carry):
        g = gi * NBUF
        for b in range(NBUF):
            c = g + b
            gather_desc(b).wait()
            store_start(c, b)
        for b in range(NBUF):
            nc = g + NBUF + b

            @pl.when(nc < NCHUNK)
            def _():
                store_wait(b)
                load_idx(nc, b)
                gather_desc(b).start()
        return carry

    lax.fori_loop(0, NGROUP, outer, 0)

    for b in range(NBUF):
        store_wait(b)


def kernel(inputs, table):
    idx = inputs.reshape(-1).astype(jnp.int32)
    return _gather_kernel(idx, table)
